# baseline (speedup 1.0000x reference)
import functools

import jax
import jax.numpy as jnp
from jax import lax
from jax.experimental import pallas as pl
from jax.experimental.pallas import tpu as pltpu
from jax.experimental.pallas import tpu_sc as plsc

_CHUNK = 400
_SUB = 100
_H = 16


def _tc_project(z_ref, wc_ref, b1_ref, a_ref, b_ref):
    r = jnp.dot(z_ref[...], wc_ref[...], preferred_element_type=jnp.float32)
    a_ref[...] = r[:, :_H] + b1_ref[...]
    b_ref[...] = r[:, _H:]


def _sc_score(nw, cpw, n_nodes, a_hbm, b_hbm, src_hbm, dst_hbm, w2_hbm, b2_hbm,
              out_hbm, idx_s, idx_d, rows_a, rows_b, out_c, w2_m, b2_v, a_sh,
              b_sh, sem):
    sid = lax.axis_index("s")
    wid = sid * 2 + lax.axis_index("c")
    seg = n_nodes // 16
    pltpu.sync_copy(a_hbm.at[pl.ds(sid * seg, seg)], a_sh.at[pl.ds(sid * seg, seg)])
    pltpu.sync_copy(b_hbm.at[pl.ds(sid * seg, seg)], b_sh.at[pl.ds(sid * seg, seg)])
    pltpu.sync_copy(w2_hbm, w2_m)
    pltpu.sync_copy(b2_hbm, b2_v)
    plsc.subcore_barrier()
    b2r = b2_v[...]
    iota16 = lax.iota(jnp.int32, 16)
    nsub = _CHUNK // _SUB

    def copy_idx(ci, parity):
        c = wid * cpw + ci
        ioff = parity * nsub
        pltpu.sync_copy(src_hbm.at[pl.ds(c * nsub, nsub)],
                        idx_s.at[pl.ds(ioff, nsub)])
        pltpu.sync_copy(dst_hbm.at[pl.ds(c * nsub, nsub)],
                        idx_d.at[pl.ds(ioff, nsub)])

    def fire_gathers(parity):
        roff = parity * _CHUNK
        ioff = parity * nsub
        for j in range(nsub):
            pltpu.async_copy(a_sh.at[idx_s.at[ioff + j]],
                             rows_a.at[pl.ds(roff + j * _SUB, _SUB)], sem)
            pltpu.async_copy(b_sh.at[idx_d.at[ioff + j]],
                             rows_b.at[pl.ds(roff + j * _SUB, _SUB)], sem)

    def wait_gathers(parity):
        roff = parity * _CHUNK
        for j in range(nsub):
            pltpu.make_async_copy(a_hbm.at[pl.ds(0, _SUB)],
                                  rows_a.at[pl.ds(roff + j * _SUB, _SUB)],
                                  sem).wait()
            pltpu.make_async_copy(a_hbm.at[pl.ds(0, _SUB)],
                                  rows_b.at[pl.ds(roff + j * _SUB, _SUB)],
                                  sem).wait()

    def compute(ci, parity):
        c = wid * cpw + ci
        roff = parity * _CHUNK

        def group_body(g, carry2):
            erow = roff + g * 16 + iota16
            accs = [b2r, jnp.zeros((16,), jnp.float32),
                    jnp.zeros((16,), jnp.float32), jnp.zeros((16,), jnp.float32)]
            for k in range(_H):
                ck = jnp.full((16,), k, jnp.int32)
                va = plsc.load_gather(rows_a, [erow, ck])
                vb = plsc.load_gather(rows_b, [erow, ck])
                accs[k % 4] = accs[k % 4] + jnp.maximum(va + vb, 0.0) * w2_m[k]
            out_c[pl.ds(g * 16, 16)] = (accs[0] + accs[1]) + (accs[2] + accs[3])
            return carry2

        lax.fori_loop(0, _CHUNK // 16, group_body, 0)
        pltpu.sync_copy(out_c, out_hbm.at[pl.ds(c * _CHUNK, _CHUNK)])

    copy_idx(0, 0)
    fire_gathers(0)

    def chunk_body(ci, carry):
        p = lax.rem(ci, 2)
        copy_idx(ci + 1, 1 - p)
        fire_gathers(1 - p)
        wait_gathers(p)
        compute(ci, p)
        return carry

    lax.fori_loop(0, cpw - 1, chunk_body, 0)
    p_last = lax.rem(jnp.int32(cpw - 1), 2)
    wait_gathers(p_last)
    compute(cpw - 1, p_last)


def kernel(z, edge_index, W1, b1, W2, b2):
    n_nodes, d = z.shape
    e = edge_index.shape[1]

    wc = jnp.concatenate([W1[:, :d].T, W1[:, d:].T], axis=1)
    a_t, b_t = pl.pallas_call(
        _tc_project,
        out_shape=[
            jax.ShapeDtypeStruct((n_nodes, _H), jnp.float32),
            jax.ShapeDtypeStruct((n_nodes, _H), jnp.float32),
        ],
    )(z, wc, b1.reshape(1, _H))

    info = plsc.get_sparse_core_info()
    nc, ns = info.num_cores, info.num_subcores
    nw = nc * ns
    cpw = -(-e // (nw * _CHUNK))
    e_pad = nw * cpw * _CHUNK

    si = edge_index[0].astype(jnp.int32)
    di = edge_index[1].astype(jnp.int32)
    pad = e_pad - e
    if pad:
        zeros = jnp.zeros((pad,), jnp.int32)
        si = jnp.concatenate([si, zeros])
        di = jnp.concatenate([di, zeros])
    si = si.reshape(e_pad // _SUB, _SUB)
    di = di.reshape(e_pad // _SUB, _SUB)

    w2m = jnp.broadcast_to(W2.reshape(_H, 1), (_H, 16)).astype(jnp.float32)
    b2v = jnp.full((_H,), b2[0], jnp.float32)

    mesh = plsc.VectorSubcoreMesh(core_axis_name="c", subcore_axis_name="s")
    score = pl.kernel(
        functools.partial(_sc_score, nw, cpw, n_nodes),
        out_type=jax.ShapeDtypeStruct((e_pad,), jnp.float32),
        mesh=mesh,
        compiler_params=pltpu.CompilerParams(
            needs_layout_passes=False, use_tc_tiling_on_sc=False),
        scratch_types=[
            pltpu.VMEM((2 * _CHUNK // _SUB, _SUB), jnp.int32),
            pltpu.VMEM((2 * _CHUNK // _SUB, _SUB), jnp.int32),
            pltpu.VMEM((2 * _CHUNK, _H), jnp.float32),
            pltpu.VMEM((2 * _CHUNK, _H), jnp.float32),
            pltpu.VMEM((_CHUNK,), jnp.float32),
            pltpu.VMEM((_H, 16), jnp.float32),
            pltpu.VMEM((_H,), jnp.float32),
            pltpu.VMEM_SHARED((n_nodes, _H), jnp.float32),
            pltpu.VMEM_SHARED((n_nodes, _H), jnp.float32),
            pltpu.SemaphoreType.DMA,
        ],
    )
    out_pad = score(a_t, b_t, si, di, w2m, b2v)
    return out_pad[:e]


# speedup vs baseline: 1.1803x; 1.1803x over previous
import functools

import jax
import jax.numpy as jnp
from jax import lax
from jax.experimental import pallas as pl
from jax.experimental.pallas import tpu as pltpu
from jax.experimental.pallas import tpu_sc as plsc

_CHUNK = 1024
_SUB = 128
_H = 16


def _tc_project(z_ref, wc_ref, b1_ref, a_ref, b_ref):
    r = jnp.dot(z_ref[...], wc_ref[...], preferred_element_type=jnp.float32)
    a_ref[...] = r[:, :_H] + b1_ref[...]
    b_ref[...] = r[:, _H:]


def _sc_score(nw, cpw, n_nodes, a_hbm, b_hbm, src_hbm, dst_hbm, w2_hbm, b2_hbm,
              out_hbm, idx_s, idx_d, rows_a, rows_b, out_c, w2_m, b2_v, a_sh,
              b_sh, sem):
    sid = lax.axis_index("s")
    wid = sid * 2 + lax.axis_index("c")
    seg = n_nodes // 16
    pltpu.async_copy(a_hbm.at[pl.ds(sid * seg, seg)],
                     a_sh.at[pl.ds(sid * seg, seg)], sem)
    pltpu.async_copy(b_hbm.at[pl.ds(sid * seg, seg)],
                     b_sh.at[pl.ds(sid * seg, seg)], sem)
    pltpu.async_copy(w2_hbm, w2_m, sem)
    pltpu.async_copy(b2_hbm, b2_v, sem)
    pltpu.make_async_copy(a_hbm.at[pl.ds(0, seg)],
                          a_sh.at[pl.ds(0, seg)], sem).wait()
    pltpu.make_async_copy(a_hbm.at[pl.ds(0, seg)],
                          b_sh.at[pl.ds(0, seg)], sem).wait()
    pltpu.make_async_copy(w2_hbm, w2_m, sem).wait()
    pltpu.make_async_copy(b2_hbm, b2_v, sem).wait()
    plsc.subcore_barrier()
    b2r = b2_v[...]
    w2bc = [w2_m[k] for k in range(_H)]
    iota16 = lax.iota(jnp.int32, 16)
    nsub = _CHUNK // _SUB

    def copy_idx(ci, parity):
        c = wid * cpw + ci
        ioff = parity * nsub
        pltpu.sync_copy(src_hbm.at[pl.ds(c * nsub, nsub)],
                        idx_s.at[pl.ds(ioff, nsub)])
        pltpu.sync_copy(dst_hbm.at[pl.ds(c * nsub, nsub)],
                        idx_d.at[pl.ds(ioff, nsub)])

    def fire_gathers(parity):
        roff = parity * _CHUNK
        ioff = parity * nsub
        for j in range(nsub):
            pltpu.async_copy(a_sh.at[idx_s.at[ioff + j]],
                             rows_a.at[pl.ds(roff + j * _SUB, _SUB)], sem)
            pltpu.async_copy(b_sh.at[idx_d.at[ioff + j]],
                             rows_b.at[pl.ds(roff + j * _SUB, _SUB)], sem)

    def wait_gathers(parity):
        roff = parity * _CHUNK
        for j in range(nsub):
            pltpu.make_async_copy(a_hbm.at[pl.ds(0, _SUB)],
                                  rows_a.at[pl.ds(roff + j * _SUB, _SUB)],
                                  sem).wait()
            pltpu.make_async_copy(a_hbm.at[pl.ds(0, _SUB)],
                                  rows_b.at[pl.ds(roff + j * _SUB, _SUB)],
                                  sem).wait()

    def compute(ci, parity):
        c = wid * cpw + ci
        roff = parity * _CHUNK

        def group_body(g, carry2):
            erow = roff + g * 16 + iota16
            accs = [b2r, jnp.zeros((16,), jnp.float32),
                    jnp.zeros((16,), jnp.float32), jnp.zeros((16,), jnp.float32)]
            for k in range(_H):
                ck = jnp.full((16,), k, jnp.int32)
                va = plsc.load_gather(rows_a, [erow, ck])
                vb = plsc.load_gather(rows_b, [erow, ck])
                accs[k % 4] = accs[k % 4] + jnp.maximum(va + vb, 0.0) * w2bc[k]
            out_c[pl.ds(g * 16, 16)] = (accs[0] + accs[1]) + (accs[2] + accs[3])
            return carry2

        lax.fori_loop(0, _CHUNK // 16, group_body, 0)
        pltpu.sync_copy(out_c, out_hbm.at[pl.ds(c * _CHUNK, _CHUNK)])

    copy_idx(0, 0)
    fire_gathers(0)

    def chunk_body(ci, carry):
        p = lax.rem(ci, 2)
        copy_idx(ci + 1, 1 - p)
        fire_gathers(1 - p)
        wait_gathers(p)
        compute(ci, p)
        return carry

    lax.fori_loop(0, cpw - 1, chunk_body, 0)
    p_last = lax.rem(jnp.int32(cpw - 1), 2)
    wait_gathers(p_last)
    compute(cpw - 1, p_last)


def kernel(z, edge_index, W1, b1, W2, b2):
    n_nodes, d = z.shape
    e = edge_index.shape[1]

    wc = jnp.concatenate([W1[:, :d].T, W1[:, d:].T], axis=1)
    a_t, b_t = pl.pallas_call(
        _tc_project,
        out_shape=[
            jax.ShapeDtypeStruct((n_nodes, _H), jnp.float32),
            jax.ShapeDtypeStruct((n_nodes, _H), jnp.float32),
        ],
    )(z, wc, b1.reshape(1, _H))

    info = plsc.get_sparse_core_info()
    nc, ns = info.num_cores, info.num_subcores
    nw = nc * ns
    cpw = -(-e // (nw * _CHUNK))
    e_pad = nw * cpw * _CHUNK

    si = edge_index[0].astype(jnp.int32)
    di = edge_index[1].astype(jnp.int32)
    pad = e_pad - e
    if pad:
        zeros = jnp.zeros((pad,), jnp.int32)
        si = jnp.concatenate([si, zeros])
        di = jnp.concatenate([di, zeros])
    si = si.reshape(e_pad // _SUB, _SUB)
    di = di.reshape(e_pad // _SUB, _SUB)

    w2m = jnp.broadcast_to(W2.reshape(_H, 1), (_H, 16)).astype(jnp.float32)
    b2v = jnp.full((_H,), b2[0], jnp.float32)

    mesh = plsc.VectorSubcoreMesh(core_axis_name="c", subcore_axis_name="s")
    score = pl.kernel(
        functools.partial(_sc_score, nw, cpw, n_nodes),
        out_type=jax.ShapeDtypeStruct((e_pad,), jnp.float32),
        mesh=mesh,
        compiler_params=pltpu.CompilerParams(
            needs_layout_passes=False, use_tc_tiling_on_sc=False),
        scratch_types=[
            pltpu.VMEM((2 * _CHUNK // _SUB, _SUB), jnp.int32),
            pltpu.VMEM((2 * _CHUNK // _SUB, _SUB), jnp.int32),
            pltpu.VMEM((2 * _CHUNK, _H), jnp.float32),
            pltpu.VMEM((2 * _CHUNK, _H), jnp.float32),
            pltpu.VMEM((_CHUNK,), jnp.float32),
            pltpu.VMEM((_H, 16), jnp.float32),
            pltpu.VMEM((_H,), jnp.float32),
            pltpu.VMEM_SHARED((n_nodes, _H), jnp.float32),
            pltpu.VMEM_SHARED((n_nodes, _H), jnp.float32),
            pltpu.SemaphoreType.DMA,
        ],
    )
    out_pad = score(a_t, b_t, si, di, w2m, b2v)
    return out_pad[:e]


# speedup vs baseline: 1.2017x; 1.0181x over previous
import functools

import jax
import jax.numpy as jnp
from jax import lax
from jax.experimental import pallas as pl
from jax.experimental.pallas import tpu as pltpu
from jax.experimental.pallas import tpu_sc as plsc

_CHUNK = 1024
_SUB = 128
_H = 16


def _tc_project(z_ref, wc_ref, b1_ref, a_ref, b_ref):
    r = jnp.dot(z_ref[...], wc_ref[...], preferred_element_type=jnp.float32)
    a_ref[...] = r[:, :_H] + b1_ref[...]
    b_ref[...] = r[:, _H:]


def _sc_score(nw, cpw, full_c, tail, n_nodes, a_hbm, b_hbm, src_hbm, dst_hbm,
              w2_hbm, b2_hbm, out_hbm, idx_s, idx_d, rows_a, rows_b, out_c,
              w2_m, b2_v, a_sh, b_sh, sem):
    sid = lax.axis_index("s")
    wid = sid * 2 + lax.axis_index("c")
    seg = n_nodes // 16
    pltpu.async_copy(a_hbm.at[pl.ds(sid * seg, seg)],
                     a_sh.at[pl.ds(sid * seg, seg)], sem)
    pltpu.async_copy(b_hbm.at[pl.ds(sid * seg, seg)],
                     b_sh.at[pl.ds(sid * seg, seg)], sem)
    pltpu.async_copy(w2_hbm, w2_m, sem)
    pltpu.async_copy(b2_hbm, b2_v, sem)
    pltpu.make_async_copy(a_hbm.at[pl.ds(0, seg)],
                          a_sh.at[pl.ds(0, seg)], sem).wait()
    pltpu.make_async_copy(a_hbm.at[pl.ds(0, seg)],
                          b_sh.at[pl.ds(0, seg)], sem).wait()
    pltpu.make_async_copy(w2_hbm, w2_m, sem).wait()
    pltpu.make_async_copy(b2_hbm, b2_v, sem).wait()
    plsc.subcore_barrier()
    b2r = b2_v[...]
    w2bc = [w2_m[k] for k in range(_H)]
    iota16 = lax.iota(jnp.int32, 16)
    nsub = _CHUNK // _SUB
    t_subs = tail // _SUB
    t_groups = tail // 16

    def copy_idx(ci, parity):
        c = wid * cpw + ci
        ioff = parity * nsub

        @pl.when(c < full_c)
        def _():
            pltpu.sync_copy(src_hbm.at[pl.ds(c * nsub, nsub)],
                            idx_s.at[pl.ds(ioff, nsub)])
            pltpu.sync_copy(dst_hbm.at[pl.ds(c * nsub, nsub)],
                            idx_d.at[pl.ds(ioff, nsub)])

        if t_subs:
            @pl.when(c == full_c)
            def _():
                pltpu.sync_copy(src_hbm.at[pl.ds(c * nsub, t_subs)],
                                idx_s.at[pl.ds(ioff, t_subs)])
                pltpu.sync_copy(dst_hbm.at[pl.ds(c * nsub, t_subs)],
                                idx_d.at[pl.ds(ioff, t_subs)])

    def fire_gathers(ci, parity):
        c = wid * cpw + ci
        roff = parity * _CHUNK
        ioff = parity * nsub

        @pl.when(c < full_c)
        def _():
            for j in range(nsub):
                pltpu.async_copy(a_sh.at[idx_s.at[ioff + j]],
                                 rows_a.at[pl.ds(roff + j * _SUB, _SUB)], sem)
                pltpu.async_copy(b_sh.at[idx_d.at[ioff + j]],
                                 rows_b.at[pl.ds(roff + j * _SUB, _SUB)], sem)

        if t_subs:
            @pl.when(c == full_c)
            def _():
                for j in range(t_subs):
                    pltpu.async_copy(a_sh.at[idx_s.at[ioff + j]],
                                     rows_a.at[pl.ds(roff + j * _SUB, _SUB)],
                                     sem)
                    pltpu.async_copy(b_sh.at[idx_d.at[ioff + j]],
                                     rows_b.at[pl.ds(roff + j * _SUB, _SUB)],
                                     sem)

    def wait_gathers(ci, parity):
        c = wid * cpw + ci
        roff = parity * _CHUNK

        @pl.when(c < full_c)
        def _():
            for j in range(nsub):
                pltpu.make_async_copy(a_hbm.at[pl.ds(0, _SUB)],
                                      rows_a.at[pl.ds(roff + j * _SUB, _SUB)],
                                      sem).wait()
                pltpu.make_async_copy(a_hbm.at[pl.ds(0, _SUB)],
                                      rows_b.at[pl.ds(roff + j * _SUB, _SUB)],
                                      sem).wait()

        if t_subs:
            @pl.when(c == full_c)
            def _():
                for j in range(t_subs):
                    pltpu.make_async_copy(a_hbm.at[pl.ds(0, _SUB)],
                                          rows_a.at[pl.ds(roff + j * _SUB,
                                                          _SUB)], sem).wait()
                    pltpu.make_async_copy(a_hbm.at[pl.ds(0, _SUB)],
                                          rows_b.at[pl.ds(roff + j * _SUB,
                                                          _SUB)], sem).wait()

    def compute(ci, parity):
        c = wid * cpw + ci
        roff = parity * _CHUNK

        def group_body(g, carry2):
            erow = roff + g * 16 + iota16
            accs = [b2r, jnp.zeros((16,), jnp.float32),
                    jnp.zeros((16,), jnp.float32), jnp.zeros((16,), jnp.float32)]
            for k in range(_H):
                ck = jnp.full((16,), k, jnp.int32)
                va = plsc.load_gather(rows_a, [erow, ck])
                vb = plsc.load_gather(rows_b, [erow, ck])
                accs[k % 4] = accs[k % 4] + jnp.maximum(va + vb, 0.0) * w2bc[k]
            out_c[pl.ds(g * 16, 16)] = (accs[0] + accs[1]) + (accs[2] + accs[3])
            return carry2

        @pl.when(c < full_c)
        def _():
            lax.fori_loop(0, _CHUNK // 16, group_body, 0)
            pltpu.sync_copy(out_c, out_hbm.at[pl.ds(c * _CHUNK, _CHUNK)])

        if tail:
            @pl.when(c == full_c)
            def _():
                lax.fori_loop(0, t_groups, group_body, 0)
                pltpu.sync_copy(out_c.at[pl.ds(0, tail)],
                                out_hbm.at[pl.ds(c * _CHUNK, tail)])

    copy_idx(0, 0)
    fire_gathers(0, 0)

    def chunk_body(ci, carry):
        p = lax.rem(ci, 2)
        copy_idx(ci + 1, 1 - p)
        fire_gathers(ci + 1, 1 - p)
        wait_gathers(ci, p)
        compute(ci, p)
        return carry

    lax.fori_loop(0, cpw - 1, chunk_body, 0)
    p_last = lax.rem(jnp.int32(cpw - 1), 2)
    wait_gathers(cpw - 1, p_last)
    compute(cpw - 1, p_last)


def kernel(z, edge_index, W1, b1, W2, b2):
    n_nodes, d = z.shape
    e = edge_index.shape[1]

    wc = jnp.concatenate([W1[:, :d].T, W1[:, d:].T], axis=1)
    a_t, b_t = pl.pallas_call(
        _tc_project,
        out_shape=[
            jax.ShapeDtypeStruct((n_nodes, _H), jnp.float32),
            jax.ShapeDtypeStruct((n_nodes, _H), jnp.float32),
        ],
    )(z, wc, b1.reshape(1, _H))

    info = plsc.get_sparse_core_info()
    nc, ns = info.num_cores, info.num_subcores
    nw = nc * ns

    e_sub = -(-e // _SUB) * _SUB
    pad = e_sub - e
    cpw = -(-e_sub // (nw * _CHUNK))
    full_c = e_sub // _CHUNK
    tail = e_sub - full_c * _CHUNK

    si = edge_index[0].astype(jnp.int32)
    di = edge_index[1].astype(jnp.int32)
    if pad:
        zeros = jnp.zeros((pad,), jnp.int32)
        si = jnp.concatenate([si, zeros])
        di = jnp.concatenate([di, zeros])
    si = si.reshape(e_sub // _SUB, _SUB)
    di = di.reshape(e_sub // _SUB, _SUB)

    w2m = jnp.broadcast_to(W2.reshape(_H, 1), (_H, 16)).astype(jnp.float32)
    b2v = jnp.full((_H,), b2[0], jnp.float32)

    mesh = plsc.VectorSubcoreMesh(core_axis_name="c", subcore_axis_name="s")
    score = pl.kernel(
        functools.partial(_sc_score, nw, cpw, full_c, tail, n_nodes),
        out_type=jax.ShapeDtypeStruct((e_sub,), jnp.float32),
        mesh=mesh,
        compiler_params=pltpu.CompilerParams(
            needs_layout_passes=False, use_tc_tiling_on_sc=False),
        scratch_types=[
            pltpu.VMEM((2 * _CHUNK // _SUB, _SUB), jnp.int32),
            pltpu.VMEM((2 * _CHUNK // _SUB, _SUB), jnp.int32),
            pltpu.VMEM((2 * _CHUNK, _H), jnp.float32),
            pltpu.VMEM((2 * _CHUNK, _H), jnp.float32),
            pltpu.VMEM((_CHUNK,), jnp.float32),
            pltpu.VMEM((_H, 16), jnp.float32),
            pltpu.VMEM((_H,), jnp.float32),
            pltpu.VMEM_SHARED((n_nodes, _H), jnp.float32),
            pltpu.VMEM_SHARED((n_nodes, _H), jnp.float32),
            pltpu.SemaphoreType.DMA,
        ],
    )
    out = score(a_t, b_t, si, di, w2m, b2v)
    if pad:
        out = out[:e]
    return out


# speedup vs baseline: 1.5857x; 1.3196x over previous
import functools

import jax
import jax.numpy as jnp
from jax import lax
from jax.experimental import pallas as pl
from jax.experimental.pallas import tpu as pltpu
from jax.experimental.pallas import tpu_sc as plsc

_CHUNK = 1024
_SUB = 128
_H = 16


def _tc_project(z_ref, wc_ref, b1_ref, a_ref, b_ref):
    r = jnp.dot(z_ref[...], wc_ref[...], preferred_element_type=jnp.float32)
    a_ref[...] = r[:, :_H] + b1_ref[...]
    b_ref[...] = r[:, _H:]


def _sc_score(nw, cpw, full_c, tail, n_nodes, a_hbm, b_hbm, src_hbm, dst_hbm,
              w2_hbm, b2_hbm, out_hbm, idx_s, idx_d, rows_s, out_c,
              w2_m, b2_v, a_sh, b_sh, sem):
    sid = lax.axis_index("s")
    wid = sid * 2 + lax.axis_index("c")
    seg = n_nodes // 16
    pltpu.async_copy(a_hbm.at[pl.ds(sid * seg, seg)],
                     a_sh.at[pl.ds(sid * seg, seg)], sem)
    pltpu.async_copy(b_hbm.at[pl.ds(sid * seg, seg)],
                     b_sh.at[pl.ds(sid * seg, seg)], sem)
    pltpu.async_copy(w2_hbm, w2_m, sem)
    pltpu.async_copy(b2_hbm, b2_v, sem)
    pltpu.make_async_copy(a_hbm.at[pl.ds(0, seg)],
                          a_sh.at[pl.ds(0, seg)], sem).wait()
    pltpu.make_async_copy(a_hbm.at[pl.ds(0, seg)],
                          b_sh.at[pl.ds(0, seg)], sem).wait()
    pltpu.make_async_copy(w2_hbm, w2_m, sem).wait()
    pltpu.make_async_copy(b2_hbm, b2_v, sem).wait()
    plsc.subcore_barrier()
    b2r = b2_v[...]
    w2bc = [w2_m[k] for k in range(_H)]
    iota16 = lax.iota(jnp.int32, 16)
    nsub = _CHUNK // _SUB
    t_subs = tail // _SUB
    t_groups = tail // 16

    def copy_idx(ci, parity):
        c = wid * cpw + ci
        ioff = parity * nsub

        @pl.when(c < full_c)
        def _():
            pltpu.sync_copy(src_hbm.at[pl.ds(c * nsub, nsub)],
                            idx_s.at[pl.ds(ioff, nsub)])
            pltpu.sync_copy(dst_hbm.at[pl.ds(c * nsub, nsub)],
                            idx_d.at[pl.ds(ioff, nsub)])

        if t_subs:
            @pl.when(c == full_c)
            def _():
                pltpu.sync_copy(src_hbm.at[pl.ds(c * nsub, t_subs)],
                                idx_s.at[pl.ds(ioff, t_subs)])
                pltpu.sync_copy(dst_hbm.at[pl.ds(c * nsub, t_subs)],
                                idx_d.at[pl.ds(ioff, t_subs)])

    def fire_a(ci, parity):
        c = wid * cpw + ci
        roff = parity * _CHUNK
        ioff = parity * nsub

        @pl.when(c < full_c)
        def _():
            for j in range(nsub):
                pltpu.async_copy(a_sh.at[idx_s.at[ioff + j]],
                                 rows_s.at[pl.ds(roff + j * _SUB, _SUB)], sem)

        if t_subs:
            @pl.when(c == full_c)
            def _():
                for j in range(t_subs):
                    pltpu.async_copy(a_sh.at[idx_s.at[ioff + j]],
                                     rows_s.at[pl.ds(roff + j * _SUB, _SUB)],
                                     sem)

    def fire_b(ci, parity):
        c = wid * cpw + ci
        roff = parity * _CHUNK
        ioff = parity * nsub

        @pl.when(c < full_c)
        def _():
            for j in range(nsub):
                pltpu.async_copy(b_sh.at[idx_d.at[ioff + j]],
                                 rows_s.at[pl.ds(roff + j * _SUB, _SUB)], sem,
                                 add=True)

        if t_subs:
            @pl.when(c == full_c)
            def _():
                for j in range(t_subs):
                    pltpu.async_copy(b_sh.at[idx_d.at[ioff + j]],
                                     rows_s.at[pl.ds(roff + j * _SUB, _SUB)],
                                     sem, add=True)

    def wait_rows(ci, parity):
        c = wid * cpw + ci
        roff = parity * _CHUNK

        @pl.when(c < full_c)
        def _():
            for j in range(nsub):
                pltpu.make_async_copy(a_hbm.at[pl.ds(0, _SUB)],
                                      rows_s.at[pl.ds(roff + j * _SUB, _SUB)],
                                      sem).wait()

        if t_subs:
            @pl.when(c == full_c)
            def _():
                for j in range(t_subs):
                    pltpu.make_async_copy(a_hbm.at[pl.ds(0, _SUB)],
                                          rows_s.at[pl.ds(roff + j * _SUB,
                                                          _SUB)], sem).wait()

    def compute(ci, parity):
        c = wid * cpw + ci
        roff = parity * _CHUNK

        def group_body(g, carry2):
            erow = roff + g * 16 + iota16
            accs = [b2r, jnp.zeros((16,), jnp.float32),
                    jnp.zeros((16,), jnp.float32), jnp.zeros((16,), jnp.float32)]
            for k in range(_H):
                ck = jnp.full((16,), k, jnp.int32)
                vs = plsc.load_gather(rows_s, [erow, ck])
                accs[k % 4] = accs[k % 4] + jnp.maximum(vs, 0.0) * w2bc[k]
            out_c[pl.ds(g * 16, 16)] = (accs[0] + accs[1]) + (accs[2] + accs[3])
            return carry2

        @pl.when(c < full_c)
        def _():
            lax.fori_loop(0, _CHUNK // 16, group_body, 0)
            pltpu.sync_copy(out_c, out_hbm.at[pl.ds(c * _CHUNK, _CHUNK)])

        if tail:
            @pl.when(c == full_c)
            def _():
                lax.fori_loop(0, t_groups, group_body, 0)
                pltpu.sync_copy(out_c.at[pl.ds(0, tail)],
                                out_hbm.at[pl.ds(c * _CHUNK, tail)])

    copy_idx(0, 0)
    fire_a(0, 0)
    if cpw >= 2:
        copy_idx(1, 1)
        fire_a(1, 1)
    wait_rows(0, 0)
    fire_b(0, 0)

    def chunk_body(ci, carry):
        p0 = lax.rem(ci, 3)
        p1 = lax.rem(ci + 1, 3)
        p2 = lax.rem(ci + 2, 3)
        copy_idx(ci + 2, p2)
        fire_a(ci + 2, p2)
        wait_rows(ci + 1, p1)
        fire_b(ci + 1, p1)
        wait_rows(ci, p0)
        compute(ci, p0)
        return carry

    if cpw >= 3:
        lax.fori_loop(0, cpw - 2, chunk_body, 0)
    if cpw >= 2:
        wait_rows(cpw - 1, (cpw - 1) % 3)
        fire_b(cpw - 1, (cpw - 1) % 3)
        wait_rows(cpw - 2, (cpw - 2) % 3)
        compute(cpw - 2, (cpw - 2) % 3)
    wait_rows(cpw - 1, (cpw - 1) % 3)
    compute(cpw - 1, (cpw - 1) % 3)


def kernel(z, edge_index, W1, b1, W2, b2):
    n_nodes, d = z.shape
    e = edge_index.shape[1]

    wc = jnp.concatenate([W1[:, :d].T, W1[:, d:].T], axis=1)
    a_t, b_t = pl.pallas_call(
        _tc_project,
        out_shape=[
            jax.ShapeDtypeStruct((n_nodes, _H), jnp.float32),
            jax.ShapeDtypeStruct((n_nodes, _H), jnp.float32),
        ],
    )(z, wc, b1.reshape(1, _H))

    info = plsc.get_sparse_core_info()
    nc, ns = info.num_cores, info.num_subcores
    nw = nc * ns

    e_sub = -(-e // _SUB) * _SUB
    pad = e_sub - e
    cpw = -(-e_sub // (nw * _CHUNK))
    full_c = e_sub // _CHUNK
    tail = e_sub - full_c * _CHUNK

    si = edge_index[0].astype(jnp.int32)
    di = edge_index[1].astype(jnp.int32)
    if pad:
        zeros = jnp.zeros((pad,), jnp.int32)
        si = jnp.concatenate([si, zeros])
        di = jnp.concatenate([di, zeros])
    si = si.reshape(e_sub // _SUB, _SUB)
    di = di.reshape(e_sub // _SUB, _SUB)

    w2m = jnp.broadcast_to(W2.reshape(_H, 1), (_H, 16)).astype(jnp.float32)
    b2v = jnp.full((_H,), b2[0], jnp.float32)

    mesh = plsc.VectorSubcoreMesh(core_axis_name="c", subcore_axis_name="s")
    score = pl.kernel(
        functools.partial(_sc_score, nw, cpw, full_c, tail, n_nodes),
        out_type=jax.ShapeDtypeStruct((e_sub,), jnp.float32),
        mesh=mesh,
        compiler_params=pltpu.CompilerParams(
            needs_layout_passes=False, use_tc_tiling_on_sc=False),
        scratch_types=[
            pltpu.VMEM((3 * _CHUNK // _SUB, _SUB), jnp.int32),
            pltpu.VMEM((3 * _CHUNK // _SUB, _SUB), jnp.int32),
            pltpu.VMEM((3 * _CHUNK, _H), jnp.float32),
            pltpu.VMEM((_CHUNK,), jnp.float32),
            pltpu.VMEM((_H, 16), jnp.float32),
            pltpu.VMEM((_H,), jnp.float32),
            pltpu.VMEM_SHARED((n_nodes, _H), jnp.float32),
            pltpu.VMEM_SHARED((n_nodes, _H), jnp.float32),
            pltpu.SemaphoreType.DMA,
        ],
    )
    out = score(a_t, b_t, si, di, w2m, b2v)
    if pad:
        out = out[:e]
    return out
